# Initial kernel scaffold; baseline (speedup 1.0000x reference)
import jax
import jax.numpy as jnp
from jax.experimental import pallas as pl


def kernel(x, edge_index, batch, conv_w, conv_b, lin1_w, lin1_b, gcn_w, gcn_b, mlp1_w, mlp1_b, mlp2_w, mlp2_b):
    raise NotImplementedError("write your pallas kernel here")


# speedup vs baseline: 4.7551x; 4.7551x over previous
import functools

import jax
import jax.numpy as jnp
from jax import lax
from jax.experimental import pallas as pl
from jax.experimental.pallas import tpu as pltpu
from jax.experimental.pallas import tpu_sc as plsc

N_NODES = 10000
NP = 10240
N_EDGES = 640000
NW = 32
TPW = 160
ICH = 32
EPT = TPW * 128
EP = NW * EPT
PAD_ROW = NP - 1
NUM_GRAPHS = 64

BA = 128
GA = NP // BA
BC = 1024
BE = 1024
ROWS_PER_TILE = NP // 16


def _cnn_body(x_ref, wrow_ref, cb_ref, l1w_ref, l1b_ref, gw_ref, out_ref):
    xr = x_ref[...]
    rows = jnp.concatenate(
        [xr[:, 0:16, :], xr[:, 1:17, :], xr[:, 2:18, :]], axis=2
    ).reshape(BA * 16, 162)
    a = jnp.dot(rows, wrow_ref[...], preferred_element_type=jnp.float32)
    a = jnp.maximum(a + cb_ref[...], 0.0)
    a5 = a.reshape(BA, 8, 2, 16, 32)
    m1 = jnp.max(a5, axis=2)
    m2 = jnp.max(m1.reshape(BA, 8, 8, 2, 32), axis=3)
    flat = m2.reshape(BA, 2048)
    feats = jnp.maximum(
        jnp.dot(flat, l1w_ref[...], preferred_element_type=jnp.float32)
        + l1b_ref[...], 0.0)
    out_ref[...] = jnp.dot(feats, gw_ref[...],
                           preferred_element_type=jnp.float32)


def _run_cnn(xt_pad, wrow, cb_cols, l1w, l1b, gw):
    return pl.pallas_call(
        _cnn_body,
        grid=(GA,),
        in_specs=[
            pl.BlockSpec((BA, 18, 54), lambda i: (i, 0, 0)),
            pl.BlockSpec((162, 512), lambda i: (0, 0)),
            pl.BlockSpec((1, 512), lambda i: (0, 0)),
            pl.BlockSpec((2048, 64), lambda i: (0, 0)),
            pl.BlockSpec((1, 64), lambda i: (0, 0)),
            pl.BlockSpec((64, 128), lambda i: (0, 0)),
        ],
        out_specs=pl.BlockSpec((BA, 128), lambda i: (i, 0)),
        out_shape=jax.ShapeDtypeStruct((NP, 128), jnp.float32),
    )(xt_pad, wrow, cb_cols, l1w, l1b, gw)


BD = 8192
ND = EP // BD


def _deg_body(dst_ref, out_ref, acc):
    pid = pl.program_id(0)

    @pl.when(pid == 0)
    def _init():
        acc[...] = jnp.zeros_like(acc)

    d = dst_ref[...]
    hi = d // 128
    lo = d - hi * 128
    oh_hi = (hi == lax.broadcasted_iota(jnp.int32, (BD, NP // 128), 1)
             ).astype(jnp.bfloat16)
    oh_lo = (lo == lax.broadcasted_iota(jnp.int32, (BD, 128), 1)
             ).astype(jnp.bfloat16)
    acc[...] += lax.dot_general(oh_hi, oh_lo, (((0,), (0,)), ((), ())),
                                preferred_element_type=jnp.float32)

    @pl.when(pid == ND - 1)
    def _fin():
        out_ref[...] = acc[...]


def _run_deg(dst2):
    return pl.pallas_call(
        _deg_body,
        grid=(ND,),
        in_specs=[pl.BlockSpec((BD, 1), lambda i: (i, 0))],
        out_specs=pl.BlockSpec((NP // 128, 128), lambda i: (0, 0)),
        out_shape=jax.ShapeDtypeStruct((NP // 128, 128), jnp.float32),
        scratch_shapes=[pltpu.VMEM((NP // 128, 128), jnp.float32)],
    )(dst2)


def _scale_body(hw_ref, deg_ref, out_ref):
    deg = deg_ref[...] + 1.0
    dinv = lax.rsqrt(deg)
    rid = (pl.program_id(0) * BC
           + lax.broadcasted_iota(jnp.int32, (BC, 1), 0))
    hs = hw_ref[...] * dinv
    out_ref[...] = jnp.where(rid < N_NODES, hs, 0.0)


def _run_scale(hw, deg):
    return pl.pallas_call(
        _scale_body,
        grid=(NP // BC,),
        in_specs=[
            pl.BlockSpec((BC, 128), lambda i: (i, 0)),
            pl.BlockSpec((BC, 1), lambda i: (i, 0)),
        ],
        out_specs=pl.BlockSpec((BC, 128), lambda i: (i, 0)),
        out_shape=jax.ShapeDtypeStruct((NP, 128), jnp.float32),
    )(hw, deg)


def _scatter_body(hs_hbm, src_hbm, dst_hbm, zeros_hbm, out_hbm,
                  sidx, didx, buf, pre_sp):
    cid = lax.axis_index("c")
    sid = lax.axis_index("s")
    wid = sid * 2 + cid
    pltpu.sync_copy(zeros_hbm, buf)
    for k in range(ROWS_PER_TILE // 128):
        pltpu.sync_copy(
            buf, pre_sp.at[pl.ds(sid * ROWS_PER_TILE + k * 128, 128)])
    plsc.subcore_barrier()

    def body(j, carry):
        off = (wid * TPW + j) * 128
        pltpu.sync_copy(src_hbm.at[pl.ds(off, 128)], sidx)
        pltpu.sync_copy(dst_hbm.at[pl.ds(off, 128)], didx)
        pltpu.sync_copy(hs_hbm.at[sidx], buf)
        pltpu.sync_copy(buf, pre_sp.at[didx], add=True)
        return carry

    lax.fori_loop(0, TPW, body, 0)
    plsc.subcore_barrier()
    pltpu.sync_copy(
        pre_sp.at[pl.ds(sid * ROWS_PER_TILE, ROWS_PER_TILE)],
        out_hbm.at[pl.ds(cid * NP + sid * ROWS_PER_TILE, ROWS_PER_TILE)])


def _run_scatter(hs, srcf, dstf, zeros128):
    mesh = plsc.VectorSubcoreMesh(core_axis_name="c", subcore_axis_name="s")
    return pl.kernel(
        _scatter_body,
        out_type=jax.ShapeDtypeStruct((2 * NP, 128), jnp.float32),
        mesh=mesh,
        scratch_types=[
            pltpu.VMEM((128,), jnp.int32),
            pltpu.VMEM((128,), jnp.int32),
            pltpu.VMEM((128, 128), jnp.float32),
            pltpu.VMEM_SHARED((NP, 128), jnp.float32),
        ],
    )(hs, srcf, dstf, zeros128)


def _pool_body(pa_ref, pb_ref, hw_ref, deg_ref, b_ref, gb_ref,
               m1w_ref, m1b_ref, m2w_ref, m2b_ref, out_ref,
               acc_s, acc_c):
    pid = pl.program_id(0)

    @pl.when(pid == 0)
    def _init():
        acc_s[...] = jnp.zeros_like(acc_s)
        acc_c[...] = jnp.zeros_like(acc_c)

    deg = deg_ref[...] + 1.0
    dinv = lax.rsqrt(deg)
    h = jnp.maximum(
        dinv * (pa_ref[...] + pb_ref[...])
        + dinv * dinv * hw_ref[...] + gb_ref[...], 0.0)
    rid = pid * BE + lax.broadcasted_iota(jnp.int32, (BE, 1), 0)
    gid = lax.broadcasted_iota(jnp.int32, (BE, NUM_GRAPHS), 1)
    oh = jnp.where((b_ref[...] == gid) & (rid < N_NODES), 1.0, 0.0)
    acc_s[...] += lax.dot_general(
        oh, h, (((0,), (0,)), ((), ())),
        preferred_element_type=jnp.float32)
    acc_c[...] += lax.dot_general(
        oh, jnp.ones_like(h), (((0,), (0,)), ((), ())),
        preferred_element_type=jnp.float32)

    @pl.when(pid == (NP // BE) - 1)
    def _fin():
        g = acc_s[...] / jnp.maximum(acc_c[...], 1.0)
        t = jnp.maximum(
            jnp.dot(g, m1w_ref[...], preferred_element_type=jnp.float32)
            + m1b_ref[...], 0.0)
        out_ref[...] = (jnp.dot(t, m2w_ref[...],
                                preferred_element_type=jnp.float32)
                        + m2b_ref[...])


def _run_pool(pa, pb, hw, deg, batch2, gb, m1w, m1b, m2w, m2b):
    blk = lambda r, c: pl.BlockSpec((BE, c), lambda i: (i, 0))
    full = lambda r, c: pl.BlockSpec((r, c), lambda i: (0, 0))
    return pl.pallas_call(
        _pool_body,
        grid=(NP // BE,),
        in_specs=[
            blk(NP, 128), blk(NP, 128), blk(NP, 128),
            blk(NP, 1), blk(NP, 1),
            full(1, 128), full(128, 64), full(1, 64),
            full(64, 2), full(1, 2),
        ],
        out_specs=pl.BlockSpec((NUM_GRAPHS, 2), lambda i: (0, 0)),
        out_shape=jax.ShapeDtypeStruct((NUM_GRAPHS, 2), jnp.float32),
        scratch_shapes=[
            pltpu.VMEM((NUM_GRAPHS, 128), jnp.float32),
            pltpu.VMEM((NUM_GRAPHS, 128), jnp.float32),
        ],
    )(pa, pb, hw, deg, batch2, gb, m1w, m1b, m2w, m2b)


def kernel(x, edge_index, batch, conv_w, conv_b, lin1_w, lin1_b,
           gcn_w, gcn_b, mlp1_w, mlp1_b, mlp2_w, mlp2_b):
    f32 = jnp.float32
    w4 = conv_w.transpose(2, 3, 1, 0)
    wrow = jnp.zeros((3, 18, 3, 16, 32), f32)
    for j in range(16):
        wrow = wrow.at[:, j:j + 3, :, j, :].set(w4)
    wrow = wrow.reshape(162, 512)
    cb_cols = jnp.tile(conv_b, 16).reshape(1, 512)
    l1w = lin1_w.reshape(32, 8, 8, 64).transpose(1, 2, 0, 3).reshape(2048, 64)

    xt = x.transpose(0, 2, 3, 1).reshape(N_NODES, 16, 48)
    xt = jnp.pad(xt, ((0, NP - N_NODES), (1, 1), (3, 3)))

    ei = edge_index.astype(jnp.int32)
    pad = jnp.full((EP - N_EDGES,), PAD_ROW, jnp.int32)
    srcf = jnp.concatenate([ei[0], pad])
    dstf = jnp.concatenate([ei[1], pad])
    batch2 = jnp.pad(batch.astype(jnp.int32),
                     (0, NP - N_NODES)).reshape(NP, 1)
    zeros128 = jnp.zeros((128, 128), f32)

    hw = _run_cnn(xt, wrow, cb_cols, l1w, lin1_b.reshape(1, 64),
                  gcn_w)
    deg = _run_deg(dstf.reshape(EP, 1)).reshape(NP, 1)
    hs = _run_scale(hw, deg)
    pre = _run_scatter(hs, srcf, dstf, zeros128)
    out = _run_pool(pre[:NP], pre[NP:], hw, deg, batch2,
                    gcn_b.reshape(1, 128), mlp1_w,
                    mlp1_b.reshape(1, 64), mlp2_w, mlp2_b.reshape(1, 2))
    return out


# speedup vs baseline: 4.9445x; 1.0398x over previous
import functools

import jax
import jax.numpy as jnp
from jax import lax
from jax.experimental import pallas as pl
from jax.experimental.pallas import tpu as pltpu
from jax.experimental.pallas import tpu_sc as plsc

N_NODES = 10000
NP = 10240
N_EDGES = 640000
NW = 32
TPW = 160
ICH = 32
EPT = TPW * 128
EP = NW * EPT
PAD_ROW = NP - 1
NUM_GRAPHS = 64

BA = 128
GA = NP // BA
BC = 1024
BE = 1024
ROWS_PER_TILE = NP // 16


def _cnn_body(x_ref, wrow_ref, cb_ref, l1w_ref, l1b_ref, gw_ref, out_ref):
    xr = x_ref[...]
    rows = jnp.concatenate(
        [xr[:, 0:16, :], xr[:, 1:17, :], xr[:, 2:18, :]], axis=2
    ).reshape(BA * 16, 162)
    a = jnp.dot(rows, wrow_ref[...], preferred_element_type=jnp.float32)
    a = jnp.maximum(a + cb_ref[...], 0.0)
    a5 = a.reshape(BA, 8, 2, 16, 32)
    m1 = jnp.max(a5, axis=2)
    m2 = jnp.max(m1.reshape(BA, 8, 8, 2, 32), axis=3)
    flat = m2.reshape(BA, 2048)
    feats = jnp.maximum(
        jnp.dot(flat, l1w_ref[...], preferred_element_type=jnp.float32)
        + l1b_ref[...], 0.0)
    out_ref[...] = jnp.dot(feats, gw_ref[...],
                           preferred_element_type=jnp.float32)


def _run_cnn(xt_pad, wrow, cb_cols, l1w, l1b, gw):
    return pl.pallas_call(
        _cnn_body,
        grid=(GA,),
        in_specs=[
            pl.BlockSpec((BA, 18, 54), lambda i: (i, 0, 0)),
            pl.BlockSpec((162, 512), lambda i: (0, 0)),
            pl.BlockSpec((1, 512), lambda i: (0, 0)),
            pl.BlockSpec((2048, 64), lambda i: (0, 0)),
            pl.BlockSpec((1, 64), lambda i: (0, 0)),
            pl.BlockSpec((64, 128), lambda i: (0, 0)),
        ],
        out_specs=pl.BlockSpec((BA, 128), lambda i: (i, 0)),
        out_shape=jax.ShapeDtypeStruct((NP, 128), jnp.float32),
    )(xt_pad, wrow, cb_cols, l1w, l1b, gw)


BD = 8192
ND = EP // BD


def _deg_body(dst_ref, out_ref, acc):
    pid = pl.program_id(0)

    @pl.when(pid == 0)
    def _init():
        acc[...] = jnp.zeros_like(acc)

    d = dst_ref[...]
    hi = d // 128
    lo = d - hi * 128
    oh_hi = (hi == lax.broadcasted_iota(jnp.int32, (BD, NP // 128), 1)
             ).astype(jnp.bfloat16)
    oh_lo = (lo == lax.broadcasted_iota(jnp.int32, (BD, 128), 1)
             ).astype(jnp.bfloat16)
    acc[...] += lax.dot_general(oh_hi, oh_lo, (((0,), (0,)), ((), ())),
                                preferred_element_type=jnp.float32)

    @pl.when(pid == ND - 1)
    def _fin():
        out_ref[...] = acc[...]


def _run_deg(dst2):
    return pl.pallas_call(
        _deg_body,
        grid=(ND,),
        in_specs=[pl.BlockSpec((BD, 1), lambda i: (i, 0))],
        out_specs=pl.BlockSpec((NP // 128, 128), lambda i: (0, 0)),
        out_shape=jax.ShapeDtypeStruct((NP // 128, 128), jnp.float32),
        scratch_shapes=[pltpu.VMEM((NP // 128, 128), jnp.float32)],
    )(dst2)


def _scale_body(hw_ref, deg_ref, out_ref):
    deg = deg_ref[...] + 1.0
    dinv = lax.rsqrt(deg)
    rid = (pl.program_id(0) * BC
           + lax.broadcasted_iota(jnp.int32, (BC, 1), 0))
    hs = hw_ref[...] * dinv
    out_ref[...] = jnp.where(rid < N_NODES, hs, 0.0)


def _run_scale(hw, deg):
    return pl.pallas_call(
        _scale_body,
        grid=(NP // BC,),
        in_specs=[
            pl.BlockSpec((BC, 128), lambda i: (i, 0)),
            pl.BlockSpec((BC, 1), lambda i: (i, 0)),
        ],
        out_specs=pl.BlockSpec((BC, 128), lambda i: (i, 0)),
        out_shape=jax.ShapeDtypeStruct((NP, 128), jnp.float32),
    )(hw, deg)


CH = 256
NCH = EPT // CH


def _scatter_body(hs_hbm, edges_hbm, zeros_hbm, out_hbm,
                  eidx, buf, pre_sp):
    cid = lax.axis_index("c")
    sid = lax.axis_index("s")
    wid = sid * 2 + cid
    pltpu.sync_copy(
        zeros_hbm, pre_sp.at[pl.ds(sid * ROWS_PER_TILE, ROWS_PER_TILE)])
    plsc.subcore_barrier()

    def body(j, carry):
        off = (wid * NCH + j) * 2 * CH
        pltpu.sync_copy(edges_hbm.at[pl.ds(off, 2 * CH)], eidx)
        pltpu.sync_copy(hs_hbm.at[eidx.at[pl.ds(0, CH)]], buf)
        pltpu.sync_copy(buf, pre_sp.at[eidx.at[pl.ds(CH, CH)]], add=True)
        return carry

    lax.fori_loop(0, NCH, body, 0)
    plsc.subcore_barrier()
    pltpu.sync_copy(
        pre_sp.at[pl.ds(sid * ROWS_PER_TILE, ROWS_PER_TILE)],
        out_hbm.at[pl.ds(cid * NP + sid * ROWS_PER_TILE, ROWS_PER_TILE)])


def _run_scatter(hs, edges, zeros640):
    mesh = plsc.VectorSubcoreMesh(core_axis_name="c", subcore_axis_name="s")
    return pl.kernel(
        _scatter_body,
        out_type=jax.ShapeDtypeStruct((2 * NP, 128), jnp.float32),
        mesh=mesh,
        scratch_types=[
            pltpu.VMEM((2 * CH,), jnp.int32),
            pltpu.VMEM((CH, 128), jnp.float32),
            pltpu.VMEM_SHARED((NP, 128), jnp.float32),
        ],
    )(hs, edges, zeros640)


def _pool_body(pa_ref, pb_ref, hw_ref, deg_ref, b_ref, gb_ref,
               m1w_ref, m1b_ref, m2w_ref, m2b_ref, out_ref,
               acc_s, acc_c):
    pid = pl.program_id(0)

    @pl.when(pid == 0)
    def _init():
        acc_s[...] = jnp.zeros_like(acc_s)
        acc_c[...] = jnp.zeros_like(acc_c)

    deg = deg_ref[...] + 1.0
    dinv = lax.rsqrt(deg)
    h = jnp.maximum(
        dinv * (pa_ref[...] + pb_ref[...])
        + dinv * dinv * hw_ref[...] + gb_ref[...], 0.0)
    rid = pid * BE + lax.broadcasted_iota(jnp.int32, (BE, 1), 0)
    gid = lax.broadcasted_iota(jnp.int32, (BE, NUM_GRAPHS), 1)
    oh = jnp.where((b_ref[...] == gid) & (rid < N_NODES), 1.0, 0.0)
    acc_s[...] += lax.dot_general(
        oh, h, (((0,), (0,)), ((), ())),
        preferred_element_type=jnp.float32)
    acc_c[...] += lax.dot_general(
        oh, jnp.ones_like(h), (((0,), (0,)), ((), ())),
        preferred_element_type=jnp.float32)

    @pl.when(pid == (NP // BE) - 1)
    def _fin():
        g = acc_s[...] / jnp.maximum(acc_c[...], 1.0)
        t = jnp.maximum(
            jnp.dot(g, m1w_ref[...], preferred_element_type=jnp.float32)
            + m1b_ref[...], 0.0)
        out_ref[...] = (jnp.dot(t, m2w_ref[...],
                                preferred_element_type=jnp.float32)
                        + m2b_ref[...])


def _run_pool(pa, pb, hw, deg, batch2, gb, m1w, m1b, m2w, m2b):
    blk = lambda r, c: pl.BlockSpec((BE, c), lambda i: (i, 0))
    full = lambda r, c: pl.BlockSpec((r, c), lambda i: (0, 0))
    return pl.pallas_call(
        _pool_body,
        grid=(NP // BE,),
        in_specs=[
            blk(NP, 128), blk(NP, 128), blk(NP, 128),
            blk(NP, 1), blk(NP, 1),
            full(1, 128), full(128, 64), full(1, 64),
            full(64, 2), full(1, 2),
        ],
        out_specs=pl.BlockSpec((NUM_GRAPHS, 2), lambda i: (0, 0)),
        out_shape=jax.ShapeDtypeStruct((NUM_GRAPHS, 2), jnp.float32),
        scratch_shapes=[
            pltpu.VMEM((NUM_GRAPHS, 128), jnp.float32),
            pltpu.VMEM((NUM_GRAPHS, 128), jnp.float32),
        ],
    )(pa, pb, hw, deg, batch2, gb, m1w, m1b, m2w, m2b)


def kernel(x, edge_index, batch, conv_w, conv_b, lin1_w, lin1_b,
           gcn_w, gcn_b, mlp1_w, mlp1_b, mlp2_w, mlp2_b):
    f32 = jnp.float32
    w4 = conv_w.transpose(2, 3, 1, 0)
    wrow = jnp.zeros((3, 18, 3, 16, 32), f32)
    for j in range(16):
        wrow = wrow.at[:, j:j + 3, :, j, :].set(w4)
    wrow = wrow.reshape(162, 512)
    cb_cols = jnp.tile(conv_b, 16).reshape(1, 512)
    l1w = lin1_w.reshape(32, 8, 8, 64).transpose(1, 2, 0, 3).reshape(2048, 64)

    xt = x.transpose(0, 2, 3, 1).reshape(N_NODES, 16, 48)
    xt = jnp.pad(xt, ((0, NP - N_NODES), (1, 1), (3, 3)))

    ei = edge_index.astype(jnp.int32)
    pad = jnp.full((2, EP - N_EDGES,), PAD_ROW, jnp.int32)
    ev = jnp.concatenate([ei, pad], axis=1)
    dstf = ev[1]
    edges = ev.reshape(2, NW, NCH, CH).transpose(1, 2, 0, 3).reshape(-1)
    batch2 = jnp.pad(batch.astype(jnp.int32),
                     (0, NP - N_NODES)).reshape(NP, 1)
    zeros640 = jnp.zeros((ROWS_PER_TILE, 128), f32)

    hw = _run_cnn(xt, wrow, cb_cols, l1w, lin1_b.reshape(1, 64),
                  gcn_w)
    deg = _run_deg(dstf.reshape(EP, 1)).reshape(NP, 1)
    hs = _run_scale(hw, deg)
    pre = _run_scatter(hs, edges, zeros640)
    out = _run_pool(pre[:NP], pre[NP:], hw, deg, batch2,
                    gcn_b.reshape(1, 128), mlp1_w,
                    mlp1_b.reshape(1, 64), mlp2_w, mlp2_b.reshape(1, 2))
    return out


# speedup vs baseline: 5.1749x; 1.0466x over previous
import functools

import jax
import jax.numpy as jnp
from jax import lax
from jax.experimental import pallas as pl
from jax.experimental.pallas import tpu as pltpu
from jax.experimental.pallas import tpu_sc as plsc

N_NODES = 10000
NP = 10240
N_EDGES = 640000
NW = 32
TPW = 160
ICH = 32
EPT = TPW * 128
EP = NW * EPT
PAD_ROW = NP - 1
NUM_GRAPHS = 64

BA = 128
GA = NP // BA
BC = 1024
BE = 1024
ROWS_PER_TILE = NP // 16


def _cnn_body(x_ref, wrow_ref, cb_ref, l1w_ref, l1b_ref, gw_ref, out_ref):
    bf16 = jnp.bfloat16
    xr = x_ref[...]
    rows = jnp.concatenate(
        [xr[:, 0:16, :], xr[:, 1:17, :], xr[:, 2:18, :]], axis=2
    ).reshape(BA * 16, 162)
    a = jnp.dot(rows.astype(bf16), wrow_ref[...],
                preferred_element_type=jnp.float32)
    a = jnp.maximum(a + cb_ref[...], 0.0)
    a5 = a.reshape(BA, 8, 2, 16, 32)
    m1 = jnp.max(a5, axis=2)
    m2 = jnp.max(m1.reshape(BA, 8, 8, 2, 32), axis=3)
    flat = m2.reshape(BA, 2048)
    feats = jnp.maximum(
        jnp.dot(flat, l1w_ref[...], preferred_element_type=jnp.float32)
        + l1b_ref[...], 0.0)
    out_ref[...] = jnp.dot(feats, gw_ref[...],
                           preferred_element_type=jnp.float32)


def _run_cnn(xt_pad, wrow, cb_cols, l1w, l1b, gw):
    return pl.pallas_call(
        _cnn_body,
        grid=(GA,),
        in_specs=[
            pl.BlockSpec((BA, 18, 54), lambda i: (i, 0, 0)),
            pl.BlockSpec((162, 512), lambda i: (0, 0)),
            pl.BlockSpec((1, 512), lambda i: (0, 0)),
            pl.BlockSpec((2048, 64), lambda i: (0, 0)),
            pl.BlockSpec((1, 64), lambda i: (0, 0)),
            pl.BlockSpec((64, 128), lambda i: (0, 0)),
        ],
        out_specs=pl.BlockSpec((BA, 128), lambda i: (i, 0)),
        out_shape=jax.ShapeDtypeStruct((NP, 128), jnp.float32),
    )(xt_pad, wrow, cb_cols, l1w, l1b, gw)


BD = 8192
ND = EP // BD


def _deg_body(dst_ref, out_ref, acc):
    pid = pl.program_id(0)

    @pl.when(pid == 0)
    def _init():
        acc[...] = jnp.zeros_like(acc)

    d = dst_ref[...]
    hi = d // 128
    lo = d - hi * 128
    oh_hi = (hi == lax.broadcasted_iota(jnp.int32, (BD, NP // 128), 1)
             ).astype(jnp.bfloat16)
    oh_lo = (lo == lax.broadcasted_iota(jnp.int32, (BD, 128), 1)
             ).astype(jnp.bfloat16)
    acc[...] += lax.dot_general(oh_hi, oh_lo, (((0,), (0,)), ((), ())),
                                preferred_element_type=jnp.float32)

    @pl.when(pid == ND - 1)
    def _fin():
        out_ref[...] = acc[...]


def _run_deg(dst2):
    return pl.pallas_call(
        _deg_body,
        grid=(ND,),
        in_specs=[pl.BlockSpec((BD, 1), lambda i: (i, 0))],
        out_specs=pl.BlockSpec((NP // 128, 128), lambda i: (0, 0)),
        out_shape=jax.ShapeDtypeStruct((NP // 128, 128), jnp.float32),
        scratch_shapes=[pltpu.VMEM((NP // 128, 128), jnp.float32)],
    )(dst2)


def _scale_body(hw_ref, deg_ref, out_ref):
    deg = deg_ref[...] + 1.0
    dinv = lax.rsqrt(deg)
    rid = (pl.program_id(0) * BC
           + lax.broadcasted_iota(jnp.int32, (BC, 1), 0))
    hs = hw_ref[...] * dinv
    out_ref[...] = jnp.where(rid < N_NODES, hs, 0.0)


def _run_scale(hw, deg):
    return pl.pallas_call(
        _scale_body,
        grid=(NP // BC,),
        in_specs=[
            pl.BlockSpec((BC, 128), lambda i: (i, 0)),
            pl.BlockSpec((BC, 1), lambda i: (i, 0)),
        ],
        out_specs=pl.BlockSpec((BC, 128), lambda i: (i, 0)),
        out_shape=jax.ShapeDtypeStruct((NP, 128), jnp.float32),
    )(hw, deg)


CH = 256
NCH = EPT // CH


def _scatter_body(hs_hbm, edges_hbm, zeros_hbm, out_hbm,
                  eidx, buf, pre_sp):
    cid = lax.axis_index("c")
    sid = lax.axis_index("s")
    wid = sid * 2 + cid
    pltpu.sync_copy(
        zeros_hbm, pre_sp.at[pl.ds(sid * ROWS_PER_TILE, ROWS_PER_TILE)])
    plsc.subcore_barrier()

    def body(j, carry):
        off = (wid * NCH + j) * 2 * CH
        pltpu.sync_copy(edges_hbm.at[pl.ds(off, 2 * CH)], eidx)
        pltpu.sync_copy(hs_hbm.at[eidx.at[pl.ds(0, CH)]], buf)
        pltpu.sync_copy(buf, pre_sp.at[eidx.at[pl.ds(CH, CH)]], add=True)
        return carry

    lax.fori_loop(0, NCH, body, 0)
    plsc.subcore_barrier()
    pltpu.sync_copy(
        pre_sp.at[pl.ds(sid * ROWS_PER_TILE, ROWS_PER_TILE)],
        out_hbm.at[pl.ds(cid * NP + sid * ROWS_PER_TILE, ROWS_PER_TILE)])


def _run_scatter(hs, edges, zeros640):
    mesh = plsc.VectorSubcoreMesh(core_axis_name="c", subcore_axis_name="s")
    return pl.kernel(
        _scatter_body,
        out_type=jax.ShapeDtypeStruct((2 * NP, 128), jnp.float32),
        mesh=mesh,
        scratch_types=[
            pltpu.VMEM((2 * CH,), jnp.int32),
            pltpu.VMEM((CH, 128), jnp.float32),
            pltpu.VMEM_SHARED((NP, 128), jnp.float32),
        ],
    )(hs, edges, zeros640)


def _pool_body(pa_ref, pb_ref, hw_ref, deg_ref, b_ref, gb_ref,
               m1w_ref, m1b_ref, m2w_ref, m2b_ref, out_ref,
               acc_s, acc_c):
    pid = pl.program_id(0)

    @pl.when(pid == 0)
    def _init():
        acc_s[...] = jnp.zeros_like(acc_s)
        acc_c[...] = jnp.zeros_like(acc_c)

    deg = deg_ref[...] + 1.0
    dinv = lax.rsqrt(deg)
    h = jnp.maximum(
        dinv * (pa_ref[...] + pb_ref[...])
        + dinv * dinv * hw_ref[...] + gb_ref[...], 0.0)
    rid = pid * BE + lax.broadcasted_iota(jnp.int32, (BE, 1), 0)
    gid = lax.broadcasted_iota(jnp.int32, (BE, NUM_GRAPHS), 1)
    oh = jnp.where((b_ref[...] == gid) & (rid < N_NODES), 1.0, 0.0)
    acc_s[...] += lax.dot_general(
        oh, h, (((0,), (0,)), ((), ())),
        preferred_element_type=jnp.float32)
    acc_c[...] += lax.dot_general(
        oh, jnp.ones_like(h), (((0,), (0,)), ((), ())),
        preferred_element_type=jnp.float32)

    @pl.when(pid == (NP // BE) - 1)
    def _fin():
        g = acc_s[...] / jnp.maximum(acc_c[...], 1.0)
        t = jnp.maximum(
            jnp.dot(g, m1w_ref[...], preferred_element_type=jnp.float32)
            + m1b_ref[...], 0.0)
        out_ref[...] = (jnp.dot(t, m2w_ref[...],
                                preferred_element_type=jnp.float32)
                        + m2b_ref[...])


def _run_pool(pa, pb, hw, deg, batch2, gb, m1w, m1b, m2w, m2b):
    blk = lambda r, c: pl.BlockSpec((BE, c), lambda i: (i, 0))
    full = lambda r, c: pl.BlockSpec((r, c), lambda i: (0, 0))
    return pl.pallas_call(
        _pool_body,
        grid=(NP // BE,),
        in_specs=[
            blk(NP, 128), blk(NP, 128), blk(NP, 128),
            blk(NP, 1), blk(NP, 1),
            full(1, 128), full(128, 64), full(1, 64),
            full(64, 2), full(1, 2),
        ],
        out_specs=pl.BlockSpec((NUM_GRAPHS, 2), lambda i: (0, 0)),
        out_shape=jax.ShapeDtypeStruct((NUM_GRAPHS, 2), jnp.float32),
        scratch_shapes=[
            pltpu.VMEM((NUM_GRAPHS, 128), jnp.float32),
            pltpu.VMEM((NUM_GRAPHS, 128), jnp.float32),
        ],
    )(pa, pb, hw, deg, batch2, gb, m1w, m1b, m2w, m2b)


def kernel(x, edge_index, batch, conv_w, conv_b, lin1_w, lin1_b,
           gcn_w, gcn_b, mlp1_w, mlp1_b, mlp2_w, mlp2_b):
    f32 = jnp.float32
    w4 = conv_w.transpose(2, 3, 1, 0)
    wrow = jnp.zeros((3, 18, 3, 16, 32), f32)
    for j in range(16):
        wrow = wrow.at[:, j:j + 3, :, j, :].set(w4)
    wrow = wrow.reshape(162, 512)
    cb_cols = jnp.tile(conv_b, 16).reshape(1, 512)
    l1w = lin1_w.reshape(32, 8, 8, 64).transpose(1, 2, 0, 3).reshape(2048, 64)

    xt = x.transpose(0, 2, 3, 1).reshape(N_NODES, 16, 48)
    xt = jnp.pad(xt, ((0, NP - N_NODES), (1, 1), (3, 3)))

    ei = edge_index.astype(jnp.int32)
    pad = jnp.full((2, EP - N_EDGES,), PAD_ROW, jnp.int32)
    ev = jnp.concatenate([ei, pad], axis=1)
    dstf = ev[1]
    edges = ev.reshape(2, NW, NCH, CH).transpose(1, 2, 0, 3).reshape(-1)
    batch2 = jnp.pad(batch.astype(jnp.int32),
                     (0, NP - N_NODES)).reshape(NP, 1)
    zeros640 = jnp.zeros((ROWS_PER_TILE, 128), f32)

    bf16 = jnp.bfloat16
    hw = _run_cnn(xt, wrow.astype(bf16), cb_cols, l1w,
                  lin1_b.reshape(1, 64), gcn_w)
    deg = _run_deg(dstf.reshape(EP, 1)).reshape(NP, 1)
    hs = _run_scale(hw, deg)
    pre = _run_scatter(hs, edges, zeros640)
    out = _run_pool(pre[:NP], pre[NP:], hw, deg, batch2,
                    gcn_b.reshape(1, 128), mlp1_w,
                    mlp1_b.reshape(1, 64), mlp2_w, mlp2_b.reshape(1, 2))
    return out
